# Initial kernel scaffold; baseline (speedup 1.0000x reference)
import jax
import jax.numpy as jnp
from jax.experimental import pallas as pl


def kernel(x, edge_index, prior_index, bias_0, bias_1):
    raise NotImplementedError("write your pallas kernel here")


# speedup vs baseline: 12.2639x; 12.2639x over previous
import dataclasses

import jax
import jax.numpy as jnp
from jax import lax
from jax.experimental import pallas as pl
from jax.experimental.pallas import tpu as pltpu
from jax.experimental.pallas import tpu_sc as plsc

_cp = pltpu.CompilerParams()
if "needs_layout_passes" in pltpu.CompilerParams.__dataclass_fields__:
    _cp = dataclasses.replace(_cp, needs_layout_passes=False)

NC = 2
NS = 16
NW = NC * NS
LANES = 16

N_NODES = 10000
N_PAD = 10240
D = 128
E_TOTAL = 480000

DEG_CHUNK = 80
EDGE_CHUNK = 120

ROWS_PER_TILE = N_PAD // NW
ROWS_PER_SCTILE = N_PAD // NS

_mesh = plsc.VectorSubcoreMesh(core_axis_name="c", subcore_axis_name="s")


def _zero16():
    return jnp.zeros((LANES,), jnp.float32)


def _fast_rsqrt(d):
    bits = plsc.bitcast(d, jnp.int32)
    bits = jnp.int32(0x5F3759DF) - (bits >> 1)
    r = plsc.bitcast(bits, jnp.float32)
    for _ in range(3):
        r = r * (1.5 - 0.5 * d * r * r)
    return jnp.where(d > 0.5, r, 0.0)


def _k0_body(col_hbm, x_hbm, dis_out, tmp_out,
             deg_sp, ones_v, col_v, z_v, deg_v, dis_v, x_v):
    c = lax.axis_index("c")
    s = lax.axis_index("s")
    w = s * NC + c

    @pl.loop(0, ROWS_PER_SCTILE // LANES)
    def _(i):
        z_v[pl.ds(i * LANES, LANES)] = _zero16()

    @pl.loop(0, DEG_CHUNK // LANES)
    def _(i):
        ones_v[pl.ds(i * LANES, LANES)] = jnp.ones((LANES,), jnp.float32)

    pltpu.sync_copy(z_v, deg_sp.at[pl.ds(s * ROWS_PER_SCTILE, ROWS_PER_SCTILE)])
    plsc.subcore_barrier()

    e_per_tile = E_TOTAL // NS
    ebase = s * e_per_tile

    @pl.loop(0, e_per_tile // DEG_CHUNK)
    def _(ci):
        pltpu.sync_copy(col_hbm.at[pl.ds(ebase + ci * DEG_CHUNK, DEG_CHUNK)], col_v)
        pltpu.sync_copy(ones_v, deg_sp.at[col_v], add=True)

    plsc.subcore_barrier()

    nbase = w * ROWS_PER_TILE
    pltpu.sync_copy(deg_sp.at[pl.ds(nbase, ROWS_PER_TILE)], deg_v)

    @pl.loop(0, ROWS_PER_TILE // LANES)
    def _(i):
        d = deg_v[pl.ds(i * LANES, LANES)]
        dis_v[pl.ds(i * LANES, LANES)] = _fast_rsqrt(d)

    pltpu.sync_copy(dis_v, dis_out.at[pl.ds(nbase, ROWS_PER_TILE)])

    @pl.loop(0, ROWS_PER_TILE // 80)
    def _(ch):
        r0 = nbase + ch * 80
        pltpu.sync_copy(x_hbm.at[pl.ds(r0, 80)], x_v)

        @pl.loop(0, 80 // LANES)
        def _(g):
            dv = dis_v[pl.ds(ch * 80 + g * LANES, LANES)]
            for r in range(LANES):
                sv = dv[r]
                row = g * LANES + r
                for j in range(D // LANES):
                    sl = pl.ds(j * LANES, LANES)
                    x_v[row, sl] = x_v[row, sl] * sv

        pltpu.sync_copy(x_v, tmp_out.at[pl.ds(r0, 80)])


_k0 = pl.kernel(
    _k0_body,
    out_type=[
        jax.ShapeDtypeStruct((N_PAD,), jnp.float32),
        jax.ShapeDtypeStruct((N_PAD, D), jnp.float32),
    ],
    mesh=_mesh,
    compiler_params=_cp,
    scratch_types=[
        pltpu.VMEM_SHARED((N_PAD,), jnp.float32),
        pltpu.VMEM((DEG_CHUNK,), jnp.float32),
        pltpu.VMEM((DEG_CHUNK,), jnp.int32),
        pltpu.VMEM((ROWS_PER_SCTILE,), jnp.float32),
        pltpu.VMEM((ROWS_PER_TILE,), jnp.float32),
        pltpu.VMEM((ROWS_PER_TILE,), jnp.float32),
        pltpu.VMEM((80, D), jnp.float32),
    ],
)


def _k1_body(src_hbm, row_hbm, col_hbm, part_out,
             acc_sp, row_v, col_v, g_v, z_v):
    c = lax.axis_index("c")
    s = lax.axis_index("s")
    w = s * NC + c

    @pl.loop(0, 128)
    def _(r):
        for j in range(D // LANES):
            z_v[r, pl.ds(j * LANES, LANES)] = _zero16()

    for q in range(ROWS_PER_SCTILE // 128):
        pltpu.sync_copy(z_v, acc_sp.at[pl.ds(s * ROWS_PER_SCTILE + q * 128, 128)])
    plsc.subcore_barrier()

    e_per_tile = E_TOTAL // NW
    ebase = w * e_per_tile

    @pl.loop(0, e_per_tile // EDGE_CHUNK)
    def _(ci):
        e0 = ebase + ci * EDGE_CHUNK
        pltpu.sync_copy(row_hbm.at[pl.ds(e0, EDGE_CHUNK)], row_v)
        pltpu.sync_copy(col_hbm.at[pl.ds(e0, EDGE_CHUNK)], col_v)
        pltpu.sync_copy(src_hbm.at[row_v], g_v)
        pltpu.sync_copy(g_v, acc_sp.at[col_v], add=True)

    plsc.subcore_barrier()

    for q in range(ROWS_PER_SCTILE // 128):
        r0 = s * ROWS_PER_SCTILE + q * 128
        pltpu.sync_copy(acc_sp.at[pl.ds(r0, 128)], z_v)
        pltpu.sync_copy(z_v, part_out.at[pl.ds(c * N_PAD + r0, 128)])


_k1 = pl.kernel(
    _k1_body,
    out_type=jax.ShapeDtypeStruct((NC * N_PAD, D), jnp.float32),
    mesh=_mesh,
    compiler_params=_cp,
    scratch_types=[
        pltpu.VMEM_SHARED((N_PAD, D), jnp.float32),
        pltpu.VMEM((EDGE_CHUNK,), jnp.int32),
        pltpu.VMEM((EDGE_CHUNK,), jnp.int32),
        pltpu.VMEM((EDGE_CHUNK, D), jnp.float32),
        pltpu.VMEM((128, D), jnp.float32),
    ],
)


def _k2_body(scale_out, part_hbm, dis_hbm, bias_hbm, o_hbm,
             p0_v, p1_v, dis_v, bias_v):
    c = lax.axis_index("c")
    s = lax.axis_index("s")
    w = s * NC + c
    nbase = w * ROWS_PER_TILE

    pltpu.sync_copy(dis_hbm.at[pl.ds(nbase, ROWS_PER_TILE)], dis_v)
    pltpu.sync_copy(bias_hbm, bias_v)

    @pl.loop(0, ROWS_PER_TILE // 80)
    def _(ch):
        r0 = nbase + ch * 80
        pltpu.sync_copy(part_hbm.at[pl.ds(r0, 80)], p0_v)
        pltpu.sync_copy(part_hbm.at[pl.ds(N_PAD + r0, 80)], p1_v)

        @pl.loop(0, 80 // LANES)
        def _(g):
            dv = dis_v[pl.ds(ch * 80 + g * LANES, LANES)]
            for r in range(LANES):
                sv = dv[r]
                row = g * LANES + r
                for j in range(D // LANES):
                    sl = pl.ds(j * LANES, LANES)
                    a = p0_v[row, sl] + p1_v[row, sl]
                    if scale_out:
                        p0_v[row, sl] = (sv * sv) * a + sv * bias_v[sl]
                    else:
                        p0_v[row, sl] = sv * a + bias_v[sl]

        pltpu.sync_copy(p0_v, o_hbm.at[pl.ds(r0, 80)])


def _make_k2(scale_out):
    return pl.kernel(
        lambda *args: _k2_body(scale_out, *args),
        out_type=jax.ShapeDtypeStruct((N_PAD, D), jnp.float32),
        mesh=_mesh,
        compiler_params=_cp,
        scratch_types=[
            pltpu.VMEM((80, D), jnp.float32),
            pltpu.VMEM((80, D), jnp.float32),
            pltpu.VMEM((ROWS_PER_TILE,), jnp.float32),
            pltpu.VMEM((D,), jnp.float32),
        ],
    )


_k2_mid = _make_k2(True)
_k2_final = _make_k2(False)


def kernel(x, edge_index, prior_index, bias_0, bias_1):
    ei = jnp.concatenate([edge_index, prior_index], axis=1)
    row = ei[0]
    col = ei[1]
    n = x.shape[0]
    xp = jnp.zeros((N_PAD, D), jnp.float32).at[:n].set(x)

    dis, tmp0 = _k0(col, xp)
    part1 = _k1(tmp0, row, col)
    tmp1 = _k2_mid(part1, dis, bias_0)
    part2 = _k1(tmp1, row, col)
    out = _k2_final(part2, dis, bias_1)
    return out[:n]
